# baseline (speedup 1.0000x reference)
import math

import numpy as np
import jax
import jax.numpy as jnp
from jax.experimental import pallas as pl

_D = 768
_B = 4
_S = 8192
_ROWS = _B * _S
_RBLK = 256
_GRID = _ROWS // _RBLK

_MIN_TH = np.float32(-math.log(0.2))
_MAX_TH = np.float32(-math.log(0.01))
_LO = np.float32(np.nextafter(np.float32(-1.0), np.float32(0.0)))
_SQRT2 = np.float32(np.sqrt(2.0))

_KS0 = np.uint32(0)
_KS1 = np.uint32(42)
_KS2 = np.uint32(0 ^ 42 ^ 0x1BD11BDA)
_ROT0 = (13, 15, 26, 6)
_ROT1 = (17, 29, 16, 24)


def _rotl(x, r):
    return (x << np.uint32(r)) | (x >> np.uint32(32 - r))


def _threefry_noise(base):
    r_io = jax.lax.broadcasted_iota(jnp.int32, (_RBLK, _D), 0)
    d_io = jax.lax.broadcasted_iota(jnp.int32, (_RBLK, _D), 1)
    x0 = jnp.zeros((_RBLK, _D), jnp.uint32)
    x1 = (base + r_io * _D + d_io).astype(jnp.uint32)

    ks = (_KS0, _KS1, _KS2)
    x0 = x0 + ks[0]
    x1 = x1 + ks[1]
    for g in range(5):
        for r in (_ROT0 if g % 2 == 0 else _ROT1):
            x0 = x0 + x1
            x1 = _rotl(x1, r)
            x1 = x0 ^ x1
        x0 = x0 + ks[(g + 1) % 3]
        x1 = x1 + ks[(g + 2) % 3] + np.uint32(g + 1)
    bits = x0 ^ x1

    fb = (bits >> np.uint32(9)) | np.uint32(0x3F800000)
    floats = pltpu_bitcast_f32(fb) - np.float32(1.0)
    u = jnp.maximum(_LO, floats * np.float32(2.0) + _LO)

    w = -jnp.log1p(-u * u)
    w1 = w - np.float32(2.5)
    p1 = np.float32(2.81022636e-08)
    for c in (3.43273939e-07, -3.5233877e-06, -4.39150654e-06, 0.00021858087,
              -0.00125372503, -0.00417768164, 0.246640727, 1.50140941):
        p1 = p1 * w1 + np.float32(c)
    w2 = jnp.sqrt(w) - np.float32(3.0)
    p2 = np.float32(-0.000200214257)
    for c in (0.000100950558, 0.00134934322, -0.00367342844, 0.00573950773,
              -0.0076224613, 0.00943887047, 1.00167406, 2.83297682):
        p2 = p2 * w2 + np.float32(c)
    p = jnp.where(w < np.float32(5.0), p1, p2)
    return _SQRT2 * (p * u)


def pltpu_bitcast_f32(x):
    return jax.lax.bitcast_convert_type(x, jnp.float32)


def _sde_kernel(t_ref, ls_ref, ptq_ref, mu_ref, lvq_ref, o_ref):
    i = pl.program_id(0)
    noise = _threefry_noise(i * (_RBLK * _D))

    theta = _MIN_TH + jax.nn.sigmoid(ptq_ref[0, :]) * (_MAX_TH - _MIN_TH)
    var_q = jnp.exp(lvq_ref[0, :])
    var_scale = np.float32(0.5) * jnp.exp(np.float32(2.0) * ls_ref[0, :]) / theta
    mu = mu_ref[0, :]

    tqt = (-theta)[None, :] * t_ref[:, :]
    mean = jnp.exp(tqt) * mu[None, :]
    var = var_scale[None, :] + jnp.exp(np.float32(2.0) * tqt) * (var_q - var_scale)[None, :]
    o_ref[:, :] = mean + jnp.sqrt(var) * noise


def kernel(input, log_sigma, param_theta_q, mu_q, log_var_q):
    t = input.reshape(_ROWS, 1)
    params = [p.reshape(1, _D) for p in (log_sigma, param_theta_q, mu_q, log_var_q)]
    out = pl.pallas_call(
        _sde_kernel,
        grid=(_GRID,),
        in_specs=[
            pl.BlockSpec((_RBLK, 1), lambda i: (i, 0)),
            pl.BlockSpec((1, _D), lambda i: (0, 0)),
            pl.BlockSpec((1, _D), lambda i: (0, 0)),
            pl.BlockSpec((1, _D), lambda i: (0, 0)),
            pl.BlockSpec((1, _D), lambda i: (0, 0)),
        ],
        out_specs=pl.BlockSpec((_RBLK, _D), lambda i: (i, 0)),
        out_shape=jax.ShapeDtypeStruct((_ROWS, _D), jnp.float32),
    )(t, *params)
    return out.reshape(_B, _S, _D)
